# Initial kernel scaffold; baseline (speedup 1.0000x reference)
import jax
import jax.numpy as jnp
from jax.experimental import pallas as pl


def kernel(x_in, edge_index, Wc0, bc0, Wc1, bc1, Wc2, bc2, Wm0, bm0, Wm1, bm1, Wm2, bm2, Wm3, bm3):
    raise NotImplementedError("write your pallas kernel here")


# speedup vs baseline: 24.2084x; 24.2084x over previous
import functools
import math

import jax
import jax.numpy as jnp
from jax import lax
from jax.experimental import pallas as pl
from jax.experimental.pallas import tpu as pltpu
from jax.experimental.pallas import tpu_sc as plsc

N = 10000
F = 128
H = 128
C = 40
E = 320000
K = 20
NP = 10112
RPT = NP // 16
EPT = 20480
EPAD = EPT * 16
NG = 10
FW = 128
_CHUNKS = [(i * 128, min(128, RPT - i * 128)) for i in range((RPT + 127) // 128)]


def _coef_table():
    import numpy as np
    t = np.zeros((32, 48), np.float32)
    for k in range(K + 1):
        if k % 2 == 0:
            s = float((-1) ** (k // 2))
            t[k, 0:16] = s * 3.0 ** k / math.factorial(k)
            t[k, 16:32] = s * 2.0 ** k / math.factorial(k)
            t[k, 32:48] = s * 1.0 ** k / math.factorial(k)
    return jnp.asarray(t)


def _sc_deg_body(dstp, deg_out, dbuf, onesb, stg, degslab):
    c = lax.axis_index("c")
    w = lax.axis_index("s")
    row0 = w * RPT
    zero16 = jnp.zeros((16,), jnp.float32)
    one16 = jnp.ones((16,), jnp.float32)

    def _fill(i, _):
        for v in range(8):
            onesb[i, pl.ds(16 * v, 16)] = one16
            stg[i, pl.ds(16 * v, 16)] = zero16
        return 0
    lax.fori_loop(0, 128, _fill, 0)

    for base, nr in _CHUNKS:
        pltpu.sync_copy(stg.at[pl.ds(0, nr)],
                        degslab.at[pl.ds(row0 + base, nr)])
    plsc.subcore_barrier()

    def _group(g, _):
        gg = (w * NG + g) * 16
        pltpu.sync_copy(dstp.at[pl.ds(gg, 16)], dbuf)

        def _chunk(j, _):
            pltpu.sync_copy(onesb, degslab.at[dbuf.at[j]], add=True)
            return 0
        lax.fori_loop(0, 16, _chunk, 0)
        return 0
    lax.fori_loop(0, NG, _group, 0)
    plsc.subcore_barrier()

    @pl.when(c == 0)
    def _():
        for base, nr in _CHUNKS:
            pltpu.sync_copy(degslab.at[pl.ds(row0 + base, nr)],
                            stg.at[pl.ds(0, nr)])
            pltpu.sync_copy(stg.at[pl.ds(0, nr)],
                            deg_out.at[pl.ds(row0 + base, nr)])


def _sc_deg(dstp):
    mesh = plsc.VectorSubcoreMesh(core_axis_name="c", subcore_axis_name="s")
    return pl.kernel(
        _sc_deg_body,
        out_type=jax.ShapeDtypeStruct((NP, FW), jnp.float32),
        mesh=mesh,
        scratch_types=[
            pltpu.VMEM((16, 128), jnp.int32),
            pltpu.VMEM((128, FW), jnp.float32),
            pltpu.VMEM((128, FW), jnp.float32),
            pltpu.VMEM_SHARED((NP, FW), jnp.float32),
        ],
    )(dstp)


def _tc1_body(x_ref, w0_ref, b0_ref, deg_ref, us_ref, acc_ref):
    z = lax.dot_general(x_ref[...], w0_ref[...], (((1,), (1,)), ((), ())),
                        preferred_element_type=jnp.float32) + b0_ref[...]
    d = jnp.maximum(deg_ref[:, 0:1], 1.0)
    dinv = lax.rsqrt(d)
    rdeg = jnp.broadcast_to(1.0 / d, (NP, 16))
    dinvr = jnp.broadcast_to(dinv, (NP, 16))
    zeros48 = jnp.zeros((NP, 48), jnp.float32)
    zeros16 = jnp.zeros((NP, 16), jnp.float32)
    for h, sl in ((0, slice(0, 64)), (1, slice(64, 128))):
        zh = z[:, sl] * dinv
        us_ref[h * NP:(h + 1) * NP, 0:64] = zh
        us_ref[h * NP:(h + 1) * NP, 64:80] = dinvr
        us_ref[h * NP:(h + 1) * NP, 80:128] = zeros48
        acc_ref[h * NP:(h + 1) * NP, 0:64] = zh
        acc_ref[h * NP:(h + 1) * NP, 64:80] = dinvr
        acc_ref[h * NP:(h + 1) * NP, 80:96] = dinvr
        acc_ref[h * NP:(h + 1) * NP, 96:112] = rdeg
        acc_ref[h * NP:(h + 1) * NP, 112:128] = zeros16


def _tc1(x_pad, W0, b0r, degrep):
    full = lambda shape: pl.BlockSpec(shape, lambda: tuple(0 for _ in shape))
    return pl.pallas_call(
        _tc1_body,
        in_specs=[full((NP, F)), full((H, F)), full((1, H)), full((NP, FW))],
        out_specs=[full((2 * NP, FW)), full((2 * NP, FW))],
        out_shape=[
            jax.ShapeDtypeStruct((2 * NP, FW), jnp.float32),
            jax.ShapeDtypeStruct((2 * NP, FW), jnp.float32),
        ],
    )(x_pad, W0, b0r, degrep)


def _sc_body(usinit, accinit, srcp, dstp, ctab,
             acc_out, us,
             sbuf, dbuf, gbuf, abuf, ctb,
             outslab, sem):
    c = lax.axis_index("c")
    w = lax.axis_index("s")
    row0 = w * RPT
    zero16 = jnp.zeros((16,), jnp.float32)

    pltpu.sync_copy(ctab, ctb)

    for base, nr in _CHUNKS:
        pltpu.sync_copy(usinit.at[pl.ds(c * NP + row0 + base, nr)],
                        gbuf.at[pl.ds(0, nr)])
        pltpu.sync_copy(gbuf.at[pl.ds(0, nr)],
                        us.at[pl.ds(c * NP + row0 + base, nr)])
        pltpu.sync_copy(accinit.at[pl.ds(c * NP + row0 + base, nr)],
                        abuf.at[pl.ds(0, nr)])
        pltpu.sync_copy(abuf.at[pl.ds(0, nr)],
                        acc_out.at[pl.ds(c * NP + row0 + base, nr)])

    def _prop(k, _):
        def _zrow(i, _):
            for v in range(8):
                gbuf[i, pl.ds(16 * v, 16)] = zero16
            return 0
        lax.fori_loop(0, 128, _zrow, 0)
        for base, nr in _CHUNKS:
            pltpu.sync_copy(gbuf.at[pl.ds(0, nr)],
                            outslab.at[pl.ds(row0 + base, nr)])
        plsc.subcore_barrier()

        def _group(g, _):
            gg = (w * NG + g) * 16
            pltpu.sync_copy(srcp.at[pl.ds(gg, 16)], sbuf)
            pltpu.sync_copy(dstp.at[pl.ds(gg, 16)], dbuf)
            coff = c * NP

            def _adj(i, _):
                for v in range(8):
                    sbuf[i, pl.ds(16 * v, 16)] = sbuf[i, pl.ds(16 * v, 16)] + coff
                return 0
            lax.fori_loop(0, 16, _adj, 0)

            def _chunk(j, _):
                pltpu.async_copy(us.at[sbuf.at[j]], gbuf, sem).wait()
                pltpu.sync_copy(gbuf, outslab.at[dbuf.at[j]], add=True)
                return 0
            lax.fori_loop(0, 16, _chunk, 0)
            return 0
        lax.fori_loop(0, NG, _group, 0)
        plsc.subcore_barrier()

        c3v = ctb[k, pl.ds(0, 16)]
        c2v = ctb[k, pl.ds(16, 16)]
        c1v = ctb[k, pl.ds(32, 16)]
        for base, nr in _CHUNKS:
            pltpu.sync_copy(outslab.at[pl.ds(row0 + base, nr)],
                            gbuf.at[pl.ds(0, nr)])
            pltpu.sync_copy(acc_out.at[pl.ds(c * NP + row0 + base, nr)],
                            abuf.at[pl.ds(0, nr)])

            def _post_row(r, _):
                rv = abuf[r, pl.ds(96, 16)]
                for v in range(4):
                    t = gbuf[r, pl.ds(16 * v, 16)] * rv
                    gbuf[r, pl.ds(16 * v, 16)] = t
                    abuf[r, pl.ds(16 * v, 16)] = (
                        abuf[r, pl.ds(16 * v, 16)] + c3v * t)
                t = gbuf[r, pl.ds(64, 16)] * rv
                gbuf[r, pl.ds(64, 16)] = t
                abuf[r, pl.ds(64, 16)] = abuf[r, pl.ds(64, 16)] + c2v * t
                abuf[r, pl.ds(80, 16)] = abuf[r, pl.ds(80, 16)] + c1v * t
                return 0
            lax.fori_loop(0, nr, _post_row, 0)
            pltpu.sync_copy(gbuf.at[pl.ds(0, nr)],
                            us.at[pl.ds(c * NP + row0 + base, nr)])
            pltpu.sync_copy(abuf.at[pl.ds(0, nr)],
                            acc_out.at[pl.ds(c * NP + row0 + base, nr)])
        return 0
    lax.fori_loop(1, K + 1, _prop, 0)


def _sc_call(usinit, accinit, srcp, dstp, ctab):
    mesh = plsc.VectorSubcoreMesh(core_axis_name="c", subcore_axis_name="s")
    f32 = jnp.float32
    return pl.kernel(
        _sc_body,
        out_type=(
            jax.ShapeDtypeStruct((2 * NP, FW), f32),
            jax.ShapeDtypeStruct((2 * NP, FW), f32),
        ),
        mesh=mesh,
        scratch_types=[
            pltpu.VMEM((16, 128), jnp.int32),
            pltpu.VMEM((16, 128), jnp.int32),
            pltpu.VMEM((128, FW), f32),
            pltpu.VMEM((128, FW), f32),
            pltpu.VMEM((32, 48), f32),
            pltpu.VMEM_SHARED((NP, FW), f32),
            pltpu.SemaphoreType.DMA,
        ],
    )(usinit, accinit, srcp, dstp, ctab)


def _tc2_body(a0_ref, a1_ref, W1_ref, W2_ref, b1_ref, b2_ref,
              Wm0_ref, bm0_ref, Wm1_ref, bm1_ref, Wm2_ref, bm2_ref,
              Wm3_ref, bm3_ref, out_ref):
    dg = lambda a, b: lax.dot_general(a, b, (((1,), (1,)), ((), ())),
                                      preferred_element_type=jnp.float32)
    a0 = a0_ref[...]
    a1 = a1_ref[...]
    sq = lax.rsqrt(a0[:, 96:97])
    S = jnp.concatenate([a0[:, 0:64], a1[:, 0:64]], axis=1) * sq
    h = dg(dg(S, W1_ref[...]), W2_ref[...])
    w2b1 = dg(b1_ref[...], W2_ref[...])
    h = h + (a0[:, 64:65] * sq) * w2b1 + (a0[:, 80:81] * sq) * b2_ref[...]
    h = jnp.maximum(dg(h, Wm0_ref[...]) + bm0_ref[...], 0.0)
    h = jnp.maximum(dg(h, Wm1_ref[...]) + bm1_ref[...], 0.0)
    h = jnp.maximum(dg(h, Wm2_ref[...]) + bm2_ref[...], 0.0)
    lg = dg(h, Wm3_ref[...]) + bm3_ref[...]
    m = jnp.max(lg, axis=1, keepdims=True)
    s = jnp.sum(jnp.exp(lg - m), axis=1, keepdims=True)
    out_ref[...] = lg - m - jnp.log(s)


def _tc2(a0, a1, W1, W2, b1r, b2r, Wm0, bm0r, Wm1, bm1r, Wm2, bm2r, Wm3, bm3r):
    full = lambda shape: pl.BlockSpec(shape, lambda i: tuple(0 for _ in shape))
    return pl.pallas_call(
        _tc2_body,
        grid=(25,),
        in_specs=[
            pl.BlockSpec((400, FW), lambda i: (i, 0)),
            pl.BlockSpec((400, FW), lambda i: (i, 0)),
            full((H, H)), full((H, H)), full((1, H)), full((1, H)),
            full((H, H)), full((1, H)), full((H, H)), full((1, H)),
            full((H, H)), full((1, H)), full((C, H)), full((1, C)),
        ],
        out_specs=pl.BlockSpec((400, C), lambda i: (i, 0)),
        out_shape=jax.ShapeDtypeStruct((N, C), jnp.float32),
    )(a0, a1, W1, W2, b1r, b2r, Wm0, bm0r, Wm1, bm1r, Wm2, bm2r, Wm3, bm3r)


def kernel(x_in, edge_index, Wc0, bc0, Wc1, bc1, Wc2, bc2,
           Wm0, bm0, Wm1, bm1, Wm2, bm2, Wm3, bm3):
    f32 = jnp.float32
    src = edge_index[0]
    dst = edge_index[1]
    npad = EPAD - E
    pad_src = (jnp.arange(npad, dtype=jnp.int32) * 1009) % N
    pad_dst = N + (jnp.arange(npad, dtype=jnp.int32) % 16)
    srcp = jnp.concatenate([src, pad_src]).reshape(EPAD // 128, 128)
    dstp = jnp.concatenate([dst, pad_dst]).reshape(EPAD // 128, 128)

    degrep = _sc_deg(dstp)

    x_pad = jnp.concatenate([x_in, jnp.zeros((NP - N, F), f32)], axis=0)
    usinit, accinit = _tc1(x_pad, Wc0, bc0.reshape(1, H), degrep)

    ctab = _coef_table()
    acc, _ = _sc_call(usinit, accinit, srcp, dstp, ctab)

    out = _tc2(acc[0:NP], acc[NP:2 * NP],
               Wc1, Wc2, bc1.reshape(1, H), bc2.reshape(1, H),
               Wm0, bm0.reshape(1, H), Wm1, bm1.reshape(1, H),
               Wm2, bm2.reshape(1, H), Wm3, bm3.reshape(1, C))
    return out


# speedup vs baseline: 31.8775x; 1.3168x over previous
import functools
import math

import jax
import jax.numpy as jnp
from jax import lax
from jax.experimental import pallas as pl
from jax.experimental.pallas import tpu as pltpu
from jax.experimental.pallas import tpu_sc as plsc

N = 10000
F = 128
H = 128
C = 40
E = 320000
K = 20
NP = 10112
RPT = NP // 16
EPT = 20480
EPAD = EPT * 16
NG = 10
FW = 128
_CHUNKS = [(i * 128, min(128, RPT - i * 128)) for i in range((RPT + 127) // 128)]


def _coef_table():
    import numpy as np
    t = np.zeros((32, 48), np.float32)
    for k in range(K + 1):
        if k % 2 == 0:
            s = float((-1) ** (k // 2))
            t[k, 0:16] = s * 3.0 ** k / math.factorial(k)
            t[k, 16:32] = s * 2.0 ** k / math.factorial(k)
            t[k, 32:48] = s * 1.0 ** k / math.factorial(k)
    return jnp.asarray(t)


def _sc_deg_body(dstp, deg_out, dbuf, onesb, stg, degslab):
    c = lax.axis_index("c")
    w = lax.axis_index("s")
    row0 = w * RPT
    zero16 = jnp.zeros((16,), jnp.float32)
    one16 = jnp.ones((16,), jnp.float32)

    def _fill(i, _):
        for v in range(8):
            onesb[i, pl.ds(16 * v, 16)] = one16
            stg[i, pl.ds(16 * v, 16)] = zero16
        return 0
    lax.fori_loop(0, 128, _fill, 0)

    for base, nr in _CHUNKS:
        pltpu.sync_copy(stg.at[pl.ds(0, nr)],
                        degslab.at[pl.ds(row0 + base, nr)])
    plsc.subcore_barrier()

    def _group(g, _):
        gg = (w * NG + g) * 16
        pltpu.sync_copy(dstp.at[pl.ds(gg, 16)], dbuf)

        def _chunk(j, _):
            pltpu.sync_copy(onesb, degslab.at[dbuf.at[j]], add=True)
            return 0
        lax.fori_loop(0, 16, _chunk, 0)
        return 0
    lax.fori_loop(0, NG, _group, 0)
    plsc.subcore_barrier()

    @pl.when(c == 0)
    def _():
        for base, nr in _CHUNKS:
            pltpu.sync_copy(degslab.at[pl.ds(row0 + base, nr)],
                            stg.at[pl.ds(0, nr)])
            pltpu.sync_copy(stg.at[pl.ds(0, nr)],
                            deg_out.at[pl.ds(row0 + base, nr)])


def _sc_deg(dstp):
    mesh = plsc.VectorSubcoreMesh(core_axis_name="c", subcore_axis_name="s")
    return pl.kernel(
        _sc_deg_body,
        out_type=jax.ShapeDtypeStruct((NP, FW), jnp.float32),
        mesh=mesh,
        scratch_types=[
            pltpu.VMEM((16, 128), jnp.int32),
            pltpu.VMEM((128, FW), jnp.float32),
            pltpu.VMEM((128, FW), jnp.float32),
            pltpu.VMEM_SHARED((NP, FW), jnp.float32),
        ],
    )(dstp)


def _tc1_body(x_ref, w0_ref, b0_ref, deg_ref, us_ref, acc_ref):
    z = lax.dot_general(x_ref[...], w0_ref[...], (((1,), (1,)), ((), ())),
                        preferred_element_type=jnp.float32) + b0_ref[...]
    d = jnp.maximum(deg_ref[:, 0:1], 1.0)
    dinv = lax.rsqrt(d)
    rdeg = jnp.broadcast_to(1.0 / d, (NP, 16))
    dinvr = jnp.broadcast_to(dinv, (NP, 16))
    zeros48 = jnp.zeros((NP, 48), jnp.float32)
    zeros16 = jnp.zeros((NP, 16), jnp.float32)
    for h, sl in ((0, slice(0, 64)), (1, slice(64, 128))):
        zh = z[:, sl] * dinv
        us_ref[h * NP:(h + 1) * NP, 0:64] = zh
        us_ref[h * NP:(h + 1) * NP, 64:80] = dinvr
        us_ref[h * NP:(h + 1) * NP, 80:128] = zeros48
        acc_ref[h * NP:(h + 1) * NP, 0:64] = zh
        acc_ref[h * NP:(h + 1) * NP, 64:80] = dinvr
        acc_ref[h * NP:(h + 1) * NP, 80:96] = dinvr
        acc_ref[h * NP:(h + 1) * NP, 96:112] = rdeg
        acc_ref[h * NP:(h + 1) * NP, 112:128] = zeros16


def _tc1(x_pad, W0, b0r, degrep):
    full = lambda shape: pl.BlockSpec(shape, lambda: tuple(0 for _ in shape))
    return pl.pallas_call(
        _tc1_body,
        in_specs=[full((NP, F)), full((H, F)), full((1, H)), full((NP, FW))],
        out_specs=[full((2 * NP, FW)), full((2 * NP, FW))],
        out_shape=[
            jax.ShapeDtypeStruct((2 * NP, FW), jnp.float32),
            jax.ShapeDtypeStruct((2 * NP, FW), jnp.float32),
        ],
    )(x_pad, W0, b0r, degrep)


def _sc_body(usinit, accinit, srcp, dstp, ctab,
             acc_out, us,
             sbuf, dbuf, gbuf, abuf, ctb,
             outslab, sem, sga, sgb, ssa, ssb):
    c = lax.axis_index("c")
    w = lax.axis_index("s")
    row0 = w * RPT
    zero16 = jnp.zeros((16,), jnp.float32)

    pltpu.sync_copy(ctab, ctb)

    for base, nr in _CHUNKS:
        pltpu.sync_copy(usinit.at[pl.ds(c * NP + row0 + base, nr)],
                        gbuf.at[pl.ds(0, nr)])
        pltpu.sync_copy(gbuf.at[pl.ds(0, nr)],
                        us.at[pl.ds(c * NP + row0 + base, nr)])
        pltpu.sync_copy(accinit.at[pl.ds(c * NP + row0 + base, nr)],
                        abuf.at[pl.ds(0, nr)])
        pltpu.sync_copy(abuf.at[pl.ds(0, nr)],
                        acc_out.at[pl.ds(c * NP + row0 + base, nr)])

    def _prop(k, _):
        def _zrow(i, _):
            for v in range(8):
                gbuf[i, pl.ds(16 * v, 16)] = zero16
            return 0
        lax.fori_loop(0, 128, _zrow, 0)
        for base, nr in _CHUNKS:
            pltpu.sync_copy(gbuf.at[pl.ds(0, nr)],
                            outslab.at[pl.ds(row0 + base, nr)])
        plsc.subcore_barrier()

        def _gather(j, buf, gsem):
            pltpu.async_copy(us.at[sbuf.at[j]], buf, gsem)

        def _wait_gather(buf, gsem):
            pltpu.make_async_copy(us.at[sbuf.at[0]], buf, gsem).wait()

        def _scatter(j, buf, ssem):
            pltpu.async_copy(buf, outslab.at[dbuf.at[j]], ssem, add=True)

        def _wait_scatter(buf, ssem):
            pltpu.make_async_copy(buf, outslab.at[dbuf.at[0]], ssem).wait()

        def _group(g, _):
            gg = (w * NG + g) * 16
            pltpu.sync_copy(srcp.at[pl.ds(gg, 16)], sbuf)
            pltpu.sync_copy(dstp.at[pl.ds(gg, 16)], dbuf)
            coff = c * NP

            def _adj(i, _):
                for v in range(8):
                    sbuf[i, pl.ds(16 * v, 16)] = sbuf[i, pl.ds(16 * v, 16)] + coff
                return 0
            lax.fori_loop(0, 16, _adj, 0)

            _gather(0, gbuf, sga)
            _wait_gather(gbuf, sga)
            _gather(1, abuf, sgb)
            _scatter(0, gbuf, ssa)

            def _pair(m, _):
                _wait_gather(abuf, sgb)
                _wait_scatter(gbuf, ssa)
                _gather(2 * m + 2, gbuf, sga)
                _scatter(2 * m + 1, abuf, ssb)
                _wait_gather(gbuf, sga)
                _wait_scatter(abuf, ssb)
                _gather(2 * m + 3, abuf, sgb)
                _scatter(2 * m + 2, gbuf, ssa)
                return 0
            lax.fori_loop(0, 7, _pair, 0)
            _wait_gather(abuf, sgb)
            _wait_scatter(gbuf, ssa)
            _scatter(15, abuf, ssb)
            _wait_scatter(abuf, ssb)
            return 0
        lax.fori_loop(0, NG, _group, 0)
        plsc.subcore_barrier()

        c3v = ctb[k, pl.ds(0, 16)]
        c2v = ctb[k, pl.ds(16, 16)]
        c1v = ctb[k, pl.ds(32, 16)]
        for base, nr in _CHUNKS:
            pltpu.sync_copy(outslab.at[pl.ds(row0 + base, nr)],
                            gbuf.at[pl.ds(0, nr)])
            pltpu.sync_copy(acc_out.at[pl.ds(c * NP + row0 + base, nr)],
                            abuf.at[pl.ds(0, nr)])

            def _post_row(r, _):
                rv = abuf[r, pl.ds(96, 16)]
                for v in range(4):
                    t = gbuf[r, pl.ds(16 * v, 16)] * rv
                    gbuf[r, pl.ds(16 * v, 16)] = t
                    abuf[r, pl.ds(16 * v, 16)] = (
                        abuf[r, pl.ds(16 * v, 16)] + c3v * t)
                t = gbuf[r, pl.ds(64, 16)] * rv
                gbuf[r, pl.ds(64, 16)] = t
                abuf[r, pl.ds(64, 16)] = abuf[r, pl.ds(64, 16)] + c2v * t
                abuf[r, pl.ds(80, 16)] = abuf[r, pl.ds(80, 16)] + c1v * t
                return 0
            lax.fori_loop(0, nr, _post_row, 0)
            pltpu.sync_copy(gbuf.at[pl.ds(0, nr)],
                            us.at[pl.ds(c * NP + row0 + base, nr)])
            pltpu.sync_copy(abuf.at[pl.ds(0, nr)],
                            acc_out.at[pl.ds(c * NP + row0 + base, nr)])
        return 0
    lax.fori_loop(1, K + 1, _prop, 0)


def _sc_call(usinit, accinit, srcp, dstp, ctab):
    mesh = plsc.VectorSubcoreMesh(core_axis_name="c", subcore_axis_name="s")
    f32 = jnp.float32
    return pl.kernel(
        _sc_body,
        out_type=(
            jax.ShapeDtypeStruct((2 * NP, FW), f32),
            jax.ShapeDtypeStruct((2 * NP, FW), f32),
        ),
        mesh=mesh,
        scratch_types=[
            pltpu.VMEM((16, 128), jnp.int32),
            pltpu.VMEM((16, 128), jnp.int32),
            pltpu.VMEM((128, FW), f32),
            pltpu.VMEM((128, FW), f32),
            pltpu.VMEM((32, 48), f32),
            pltpu.VMEM_SHARED((NP, FW), f32),
            pltpu.SemaphoreType.DMA,
            pltpu.SemaphoreType.DMA,
            pltpu.SemaphoreType.DMA,
            pltpu.SemaphoreType.DMA,
            pltpu.SemaphoreType.DMA,
        ],
    )(usinit, accinit, srcp, dstp, ctab)


def _tc2_body(a0_ref, a1_ref, W1_ref, W2_ref, b1_ref, b2_ref,
              Wm0_ref, bm0_ref, Wm1_ref, bm1_ref, Wm2_ref, bm2_ref,
              Wm3_ref, bm3_ref, out_ref):
    dg = lambda a, b: lax.dot_general(a, b, (((1,), (1,)), ((), ())),
                                      preferred_element_type=jnp.float32)
    a0 = a0_ref[...]
    a1 = a1_ref[...]
    sq = lax.rsqrt(a0[:, 96:97])
    S = jnp.concatenate([a0[:, 0:64], a1[:, 0:64]], axis=1) * sq
    h = dg(dg(S, W1_ref[...]), W2_ref[...])
    w2b1 = dg(b1_ref[...], W2_ref[...])
    h = h + (a0[:, 64:65] * sq) * w2b1 + (a0[:, 80:81] * sq) * b2_ref[...]
    h = jnp.maximum(dg(h, Wm0_ref[...]) + bm0_ref[...], 0.0)
    h = jnp.maximum(dg(h, Wm1_ref[...]) + bm1_ref[...], 0.0)
    h = jnp.maximum(dg(h, Wm2_ref[...]) + bm2_ref[...], 0.0)
    lg = dg(h, Wm3_ref[...]) + bm3_ref[...]
    m = jnp.max(lg, axis=1, keepdims=True)
    s = jnp.sum(jnp.exp(lg - m), axis=1, keepdims=True)
    out_ref[...] = lg - m - jnp.log(s)


def _tc2(a0, a1, W1, W2, b1r, b2r, Wm0, bm0r, Wm1, bm1r, Wm2, bm2r, Wm3, bm3r):
    full = lambda shape: pl.BlockSpec(shape, lambda i: tuple(0 for _ in shape))
    return pl.pallas_call(
        _tc2_body,
        grid=(25,),
        in_specs=[
            pl.BlockSpec((400, FW), lambda i: (i, 0)),
            pl.BlockSpec((400, FW), lambda i: (i, 0)),
            full((H, H)), full((H, H)), full((1, H)), full((1, H)),
            full((H, H)), full((1, H)), full((H, H)), full((1, H)),
            full((H, H)), full((1, H)), full((C, H)), full((1, C)),
        ],
        out_specs=pl.BlockSpec((400, C), lambda i: (i, 0)),
        out_shape=jax.ShapeDtypeStruct((N, C), jnp.float32),
    )(a0, a1, W1, W2, b1r, b2r, Wm0, bm0r, Wm1, bm1r, Wm2, bm2r, Wm3, bm3r)


def kernel(x_in, edge_index, Wc0, bc0, Wc1, bc1, Wc2, bc2,
           Wm0, bm0, Wm1, bm1, Wm2, bm2, Wm3, bm3):
    f32 = jnp.float32
    src = edge_index[0]
    dst = edge_index[1]
    npad = EPAD - E
    pad_src = (jnp.arange(npad, dtype=jnp.int32) * 1009) % N
    pad_dst = N + (jnp.arange(npad, dtype=jnp.int32) % 16)
    srcp = jnp.concatenate([src, pad_src]).reshape(EPAD // 128, 128)
    dstp = jnp.concatenate([dst, pad_dst]).reshape(EPAD // 128, 128)

    degrep = _sc_deg(dstp)

    x_pad = jnp.concatenate([x_in, jnp.zeros((NP - N, F), f32)], axis=0)
    usinit, accinit = _tc1(x_pad, Wc0, bc0.reshape(1, H), degrep)

    ctab = _coef_table()
    acc, _ = _sc_call(usinit, accinit, srcp, dstp, ctab)

    out = _tc2(acc[0:NP], acc[NP:2 * NP],
               Wc1, Wc2, bc1.reshape(1, H), bc2.reshape(1, H),
               Wm0, bm0.reshape(1, H), Wm1, bm1.reshape(1, H),
               Wm2, bm2.reshape(1, H), Wm3, bm3.reshape(1, C))
    return out


# speedup vs baseline: 33.4222x; 1.0485x over previous
import functools
import math

import jax
import jax.numpy as jnp
from jax import lax
from jax.experimental import pallas as pl
from jax.experimental.pallas import tpu as pltpu
from jax.experimental.pallas import tpu_sc as plsc

N = 10000
F = 128
H = 128
C = 40
E = 320000
K = 20
NP = 10112
RPT = NP // 16
EPT = 20480
EPAD = EPT * 16
NG = 10
FW = 128
_CHUNKS = [(i * 128, min(128, RPT - i * 128)) for i in range((RPT + 127) // 128)]


def _coef_table():
    import numpy as np
    t = np.zeros((32, 48), np.float32)
    for k in range(K + 1):
        if k % 2 == 0:
            s = float((-1) ** (k // 2))
            t[k, 0:16] = s * 3.0 ** k / math.factorial(k)
            t[k, 16:32] = s * 2.0 ** k / math.factorial(k)
            t[k, 32:48] = s * 1.0 ** k / math.factorial(k)
    return jnp.asarray(t)


def _sc_deg_body(dstp, deg_out, dbuf, onesb, stg, degslab):
    c = lax.axis_index("c")
    w = lax.axis_index("s")
    row0 = w * RPT
    zero16 = jnp.zeros((16,), jnp.float32)
    one16 = jnp.ones((16,), jnp.float32)

    def _fill(i, _):
        for v in range(8):
            onesb[i, pl.ds(16 * v, 16)] = one16
            stg[i, pl.ds(16 * v, 16)] = zero16
        return 0
    lax.fori_loop(0, 128, _fill, 0)

    for base, nr in _CHUNKS:
        pltpu.sync_copy(stg.at[pl.ds(0, nr)],
                        degslab.at[pl.ds(row0 + base, nr)])
    plsc.subcore_barrier()

    def _group(g, _):
        gg = (w * NG + g) * 16
        pltpu.sync_copy(dstp.at[pl.ds(gg, 16)], dbuf)

        def _chunk(j, _):
            pltpu.sync_copy(onesb, degslab.at[dbuf.at[j]], add=True)
            return 0
        lax.fori_loop(0, 16, _chunk, 0)
        return 0
    lax.fori_loop(0, NG, _group, 0)
    plsc.subcore_barrier()

    @pl.when(c == 0)
    def _():
        for base, nr in _CHUNKS:
            pltpu.sync_copy(degslab.at[pl.ds(row0 + base, nr)],
                            stg.at[pl.ds(0, nr)])
            pltpu.sync_copy(stg.at[pl.ds(0, nr)],
                            deg_out.at[pl.ds(row0 + base, nr)])


def _sc_deg(dstp):
    mesh = plsc.VectorSubcoreMesh(core_axis_name="c", subcore_axis_name="s")
    return pl.kernel(
        _sc_deg_body,
        out_type=jax.ShapeDtypeStruct((NP, FW), jnp.float32),
        mesh=mesh,
        scratch_types=[
            pltpu.VMEM((16, 128), jnp.int32),
            pltpu.VMEM((128, FW), jnp.float32),
            pltpu.VMEM((128, FW), jnp.float32),
            pltpu.VMEM_SHARED((NP, FW), jnp.float32),
        ],
    )(dstp)


def _tc1_body(x_ref, w0_ref, b0_ref, deg_ref, us_ref, acc_ref):
    z = lax.dot_general(x_ref[...], w0_ref[...], (((1,), (1,)), ((), ())),
                        preferred_element_type=jnp.float32) + b0_ref[...]
    d = jnp.maximum(deg_ref[:, 0:1], 1.0)
    dinv = lax.rsqrt(d)
    rdeg = jnp.broadcast_to(1.0 / d, (NP, 16))
    dinvr = jnp.broadcast_to(dinv, (NP, 16))
    zeros48 = jnp.zeros((NP, 48), jnp.float32)
    zeros16 = jnp.zeros((NP, 16), jnp.float32)
    for h, sl in ((0, slice(0, 64)), (1, slice(64, 128))):
        zh = z[:, sl] * dinv
        us_ref[h * NP:(h + 1) * NP, 0:64] = zh
        us_ref[h * NP:(h + 1) * NP, 64:80] = dinvr
        us_ref[h * NP:(h + 1) * NP, 80:128] = zeros48
        acc_ref[h * NP:(h + 1) * NP, 0:64] = zh
        acc_ref[h * NP:(h + 1) * NP, 64:80] = dinvr
        acc_ref[h * NP:(h + 1) * NP, 80:96] = dinvr
        acc_ref[h * NP:(h + 1) * NP, 96:112] = rdeg
        acc_ref[h * NP:(h + 1) * NP, 112:128] = zeros16


def _tc1(x_pad, W0, b0r, degrep):
    full = lambda shape: pl.BlockSpec(shape, lambda: tuple(0 for _ in shape))
    return pl.pallas_call(
        _tc1_body,
        in_specs=[full((NP, F)), full((H, F)), full((1, H)), full((NP, FW))],
        out_specs=[full((2 * NP, FW)), full((2 * NP, FW))],
        out_shape=[
            jax.ShapeDtypeStruct((2 * NP, FW), jnp.float32),
            jax.ShapeDtypeStruct((2 * NP, FW), jnp.float32),
        ],
    )(x_pad, W0, b0r, degrep)


def _sc_body(usinit, accinit, srcp, dstp, ctab,
             acc_out, us,
             sbuf, dbuf, sbuf2, dbuf2, gbuf, abuf, ctb,
             outslab, sem, sga, sgb, ssa, ssb):
    c = lax.axis_index("c")
    w = lax.axis_index("s")
    row0 = w * RPT
    zero16 = jnp.zeros((16,), jnp.float32)

    pltpu.sync_copy(ctab, ctb)

    for base, nr in _CHUNKS:
        pltpu.sync_copy(usinit.at[pl.ds(c * NP + row0 + base, nr)],
                        gbuf.at[pl.ds(0, nr)])
        pltpu.sync_copy(gbuf.at[pl.ds(0, nr)],
                        us.at[pl.ds(c * NP + row0 + base, nr)])
        pltpu.sync_copy(accinit.at[pl.ds(c * NP + row0 + base, nr)],
                        abuf.at[pl.ds(0, nr)])
        pltpu.sync_copy(abuf.at[pl.ds(0, nr)],
                        acc_out.at[pl.ds(c * NP + row0 + base, nr)])

    def _prop(k, _):
        def _zrow(i, _):
            for v in range(8):
                gbuf[i, pl.ds(16 * v, 16)] = zero16
            return 0
        lax.fori_loop(0, 128, _zrow, 0)
        for base, nr in _CHUNKS:
            pltpu.async_copy(gbuf.at[pl.ds(0, nr)],
                             outslab.at[pl.ds(row0 + base, nr)], sem)
        for base, nr in _CHUNKS:
            pltpu.make_async_copy(gbuf.at[pl.ds(0, nr)],
                                  outslab.at[pl.ds(row0 + base, nr)], sem).wait()
        plsc.subcore_barrier()

        def _gather(sidx, j, buf, gsem):
            pltpu.async_copy(us.at[sidx.at[j]], buf, gsem)

        def _wait_gather(sidx, buf, gsem):
            pltpu.make_async_copy(us.at[sidx.at[0]], buf, gsem).wait()

        def _scatter(didx, j, buf, ssem):
            pltpu.async_copy(buf, outslab.at[didx.at[j]], ssem, add=True)

        def _wait_scatter(didx, buf, ssem):
            pltpu.make_async_copy(buf, outslab.at[didx.at[0]], ssem).wait()

        coff = c * NP

        def _adj(sidx):
            def _row(i, _):
                for v in range(8):
                    sidx[i, pl.ds(16 * v, 16)] = sidx[i, pl.ds(16 * v, 16)] + coff
                return 0
            lax.fori_loop(0, 16, _row, 0)

        def _ring16(sidx, didx):
            _gather(sidx, 0, gbuf, sga)
            _wait_gather(sidx, gbuf, sga)
            _gather(sidx, 1, abuf, sgb)
            _scatter(didx, 0, gbuf, ssa)

            def _pair(m, _):
                _wait_gather(sidx, abuf, sgb)
                _wait_scatter(didx, gbuf, ssa)
                _gather(sidx, 2 * m + 2, gbuf, sga)
                _scatter(didx, 2 * m + 1, abuf, ssb)
                _wait_gather(sidx, gbuf, sga)
                _wait_scatter(didx, abuf, ssb)
                _gather(sidx, 2 * m + 3, abuf, sgb)
                _scatter(didx, 2 * m + 2, gbuf, ssa)
                return 0
            lax.fori_loop(0, 7, _pair, 0)
            _wait_gather(sidx, abuf, sgb)
            _wait_scatter(didx, gbuf, ssa)
            _scatter(didx, 15, abuf, ssb)
            _wait_scatter(didx, abuf, ssb)

        g0 = w * NG * 16
        pltpu.sync_copy(srcp.at[pl.ds(g0, 16)], sbuf)
        pltpu.sync_copy(dstp.at[pl.ds(g0, 16)], dbuf)

        def _gpair(p, _):
            ggb = (w * NG + 2 * p + 1) * 16
            pltpu.async_copy(srcp.at[pl.ds(ggb, 16)], sbuf2, sem)
            pltpu.async_copy(dstp.at[pl.ds(ggb, 16)], dbuf2, sem)
            _adj(sbuf)
            _ring16(sbuf, dbuf)
            pltpu.make_async_copy(srcp.at[pl.ds(ggb, 16)], sbuf2, sem).wait()
            pltpu.make_async_copy(dstp.at[pl.ds(ggb, 16)], dbuf2, sem).wait()

            @pl.when(p < NG // 2 - 1)
            def _():
                ggc = (w * NG + 2 * p + 2) * 16
                pltpu.async_copy(srcp.at[pl.ds(ggc, 16)], sbuf, sem)
                pltpu.async_copy(dstp.at[pl.ds(ggc, 16)], dbuf, sem)
            _adj(sbuf2)
            _ring16(sbuf2, dbuf2)

            @pl.when(p < NG // 2 - 1)
            def _():
                ggc = (w * NG + 2 * p + 2) * 16
                pltpu.make_async_copy(srcp.at[pl.ds(ggc, 16)], sbuf, sem).wait()
                pltpu.make_async_copy(dstp.at[pl.ds(ggc, 16)], dbuf, sem).wait()
            return 0
        lax.fori_loop(0, NG // 2, _gpair, 0)
        plsc.subcore_barrier()

        c3v = ctb[k, pl.ds(0, 16)]
        c2v = ctb[k, pl.ds(16, 16)]
        c1v = ctb[k, pl.ds(32, 16)]
        for base, nr in _CHUNKS:
            pltpu.async_copy(outslab.at[pl.ds(row0 + base, nr)],
                             gbuf.at[pl.ds(0, nr)], sga)
            pltpu.async_copy(acc_out.at[pl.ds(c * NP + row0 + base, nr)],
                             abuf.at[pl.ds(0, nr)], sgb)
            pltpu.make_async_copy(outslab.at[pl.ds(row0 + base, nr)],
                                  gbuf.at[pl.ds(0, nr)], sga).wait()
            pltpu.make_async_copy(acc_out.at[pl.ds(c * NP + row0 + base, nr)],
                                  abuf.at[pl.ds(0, nr)], sgb).wait()

            def _post_row(r, _):
                rv = abuf[r, pl.ds(96, 16)]
                for v in range(4):
                    t = gbuf[r, pl.ds(16 * v, 16)] * rv
                    gbuf[r, pl.ds(16 * v, 16)] = t
                    abuf[r, pl.ds(16 * v, 16)] = (
                        abuf[r, pl.ds(16 * v, 16)] + c3v * t)
                t = gbuf[r, pl.ds(64, 16)] * rv
                gbuf[r, pl.ds(64, 16)] = t
                abuf[r, pl.ds(64, 16)] = abuf[r, pl.ds(64, 16)] + c2v * t
                abuf[r, pl.ds(80, 16)] = abuf[r, pl.ds(80, 16)] + c1v * t
                return 0
            lax.fori_loop(0, nr, _post_row, 0)
            pltpu.async_copy(gbuf.at[pl.ds(0, nr)],
                             us.at[pl.ds(c * NP + row0 + base, nr)], ssa)
            pltpu.async_copy(abuf.at[pl.ds(0, nr)],
                             acc_out.at[pl.ds(c * NP + row0 + base, nr)], ssb)
            pltpu.make_async_copy(gbuf.at[pl.ds(0, nr)],
                                  us.at[pl.ds(c * NP + row0 + base, nr)], ssa).wait()
            pltpu.make_async_copy(abuf.at[pl.ds(0, nr)],
                                  acc_out.at[pl.ds(c * NP + row0 + base, nr)], ssb).wait()
        return 0
    lax.fori_loop(1, K + 1, _prop, 0)


def _sc_call(usinit, accinit, srcp, dstp, ctab):
    mesh = plsc.VectorSubcoreMesh(core_axis_name="c", subcore_axis_name="s")
    f32 = jnp.float32
    return pl.kernel(
        _sc_body,
        out_type=(
            jax.ShapeDtypeStruct((2 * NP, FW), f32),
            jax.ShapeDtypeStruct((2 * NP, FW), f32),
        ),
        mesh=mesh,
        scratch_types=[
            pltpu.VMEM((16, 128), jnp.int32),
            pltpu.VMEM((16, 128), jnp.int32),
            pltpu.VMEM((16, 128), jnp.int32),
            pltpu.VMEM((16, 128), jnp.int32),
            pltpu.VMEM((128, FW), f32),
            pltpu.VMEM((128, FW), f32),
            pltpu.VMEM((32, 48), f32),
            pltpu.VMEM_SHARED((NP, FW), f32),
            pltpu.SemaphoreType.DMA,
            pltpu.SemaphoreType.DMA,
            pltpu.SemaphoreType.DMA,
            pltpu.SemaphoreType.DMA,
            pltpu.SemaphoreType.DMA,
        ],
    )(usinit, accinit, srcp, dstp, ctab)


def _tc2_body(a0_ref, a1_ref, W1_ref, W2_ref, b1_ref, b2_ref,
              Wm0_ref, bm0_ref, Wm1_ref, bm1_ref, Wm2_ref, bm2_ref,
              Wm3_ref, bm3_ref, out_ref):
    dg = lambda a, b: lax.dot_general(a, b, (((1,), (1,)), ((), ())),
                                      preferred_element_type=jnp.float32)
    a0 = a0_ref[...]
    a1 = a1_ref[...]
    sq = lax.rsqrt(a0[:, 96:97])
    S = jnp.concatenate([a0[:, 0:64], a1[:, 0:64]], axis=1) * sq
    h = dg(dg(S, W1_ref[...]), W2_ref[...])
    w2b1 = dg(b1_ref[...], W2_ref[...])
    h = h + (a0[:, 64:65] * sq) * w2b1 + (a0[:, 80:81] * sq) * b2_ref[...]
    h = jnp.maximum(dg(h, Wm0_ref[...]) + bm0_ref[...], 0.0)
    h = jnp.maximum(dg(h, Wm1_ref[...]) + bm1_ref[...], 0.0)
    h = jnp.maximum(dg(h, Wm2_ref[...]) + bm2_ref[...], 0.0)
    lg = dg(h, Wm3_ref[...]) + bm3_ref[...]
    m = jnp.max(lg, axis=1, keepdims=True)
    s = jnp.sum(jnp.exp(lg - m), axis=1, keepdims=True)
    out_ref[...] = lg - m - jnp.log(s)


def _tc2(a0, a1, W1, W2, b1r, b2r, Wm0, bm0r, Wm1, bm1r, Wm2, bm2r, Wm3, bm3r):
    full = lambda shape: pl.BlockSpec(shape, lambda i: tuple(0 for _ in shape))
    return pl.pallas_call(
        _tc2_body,
        grid=(25,),
        in_specs=[
            pl.BlockSpec((400, FW), lambda i: (i, 0)),
            pl.BlockSpec((400, FW), lambda i: (i, 0)),
            full((H, H)), full((H, H)), full((1, H)), full((1, H)),
            full((H, H)), full((1, H)), full((H, H)), full((1, H)),
            full((H, H)), full((1, H)), full((C, H)), full((1, C)),
        ],
        out_specs=pl.BlockSpec((400, C), lambda i: (i, 0)),
        out_shape=jax.ShapeDtypeStruct((N, C), jnp.float32),
    )(a0, a1, W1, W2, b1r, b2r, Wm0, bm0r, Wm1, bm1r, Wm2, bm2r, Wm3, bm3r)


def kernel(x_in, edge_index, Wc0, bc0, Wc1, bc1, Wc2, bc2,
           Wm0, bm0, Wm1, bm1, Wm2, bm2, Wm3, bm3):
    f32 = jnp.float32
    src = edge_index[0]
    dst = edge_index[1]
    npad = EPAD - E
    pad_src = (jnp.arange(npad, dtype=jnp.int32) * 1009) % N
    pad_dst = N + (jnp.arange(npad, dtype=jnp.int32) % 16)
    srcp = jnp.concatenate([src, pad_src]).reshape(EPAD // 128, 128)
    dstp = jnp.concatenate([dst, pad_dst]).reshape(EPAD // 128, 128)

    degrep = _sc_deg(dstp)

    x_pad = jnp.concatenate([x_in, jnp.zeros((NP - N, F), f32)], axis=0)
    usinit, accinit = _tc1(x_pad, Wc0, bc0.reshape(1, H), degrep)

    ctab = _coef_table()
    acc, _ = _sc_call(usinit, accinit, srcp, dstp, ctab)

    out = _tc2(acc[0:NP], acc[NP:2 * NP],
               Wc1, Wc2, bc1.reshape(1, H), bc2.reshape(1, H),
               Wm0, bm0.reshape(1, H), Wm1, bm1.reshape(1, H),
               Wm2, bm2.reshape(1, H), Wm3, bm3.reshape(1, C))
    return out


# speedup vs baseline: 36.8659x; 1.1030x over previous
import functools
import math

import jax
import jax.numpy as jnp
from jax import lax
from jax.experimental import pallas as pl
from jax.experimental.pallas import tpu as pltpu
from jax.experimental.pallas import tpu_sc as plsc

N = 10000
F = 128
H = 128
C = 40
E = 320000
K = 20
NP = 10112
RPT = NP // 16
CH = 112
EPT = 21504
EPAD = EPT * 16
NG = 12
FW = 128
_CHUNKS = [(i * CH, min(CH, RPT - i * CH)) for i in range((RPT + CH - 1) // CH)]


def _coef_table():
    import numpy as np
    t = np.zeros((16, 128), np.float32)
    for k in range(K + 1):
        if k % 2 == 0:
            s = float((-1) ** (k // 2))
            co = 48 * (k // 16)
            t[k % 16, co + 0:co + 16] = s * 3.0 ** k / math.factorial(k)
            t[k % 16, co + 16:co + 32] = s * 2.0 ** k / math.factorial(k)
            t[k % 16, co + 32:co + 48] = s * 1.0 ** k / math.factorial(k)
    return jnp.asarray(t)


def _sc_deg_body(dstp, deg_out, dbuf, onesb, stg, degslab):
    c = lax.axis_index("c")
    w = lax.axis_index("s")
    row0 = w * RPT
    zero16 = jnp.zeros((16,), jnp.float32)
    one16 = jnp.ones((16,), jnp.float32)

    def _fill(i, _):
        for v in range(8):
            onesb[i, pl.ds(16 * v, 16)] = one16
            stg[i, pl.ds(16 * v, 16)] = zero16
        return 0
    lax.fori_loop(0, 128, _fill, 0)

    for base, nr in _CHUNKS:
        pltpu.sync_copy(stg.at[pl.ds(0, nr)],
                        degslab.at[pl.ds(row0 + base, nr)])
    plsc.subcore_barrier()

    def _group(g, _):
        gg = (w * NG + g) * 16
        pltpu.sync_copy(dstp.at[pl.ds(gg, 16)], dbuf)

        def _chunk(j, _):
            pltpu.sync_copy(onesb.at[pl.ds(0, CH)], degslab.at[dbuf.at[j]],
                            add=True)
            return 0
        lax.fori_loop(0, 16, _chunk, 0)
        return 0
    lax.fori_loop(0, NG, _group, 0)
    plsc.subcore_barrier()

    @pl.when(c == 0)
    def _():
        for base, nr in _CHUNKS:
            pltpu.sync_copy(degslab.at[pl.ds(row0 + base, nr)],
                            stg.at[pl.ds(0, nr)])
            pltpu.sync_copy(stg.at[pl.ds(0, nr)],
                            deg_out.at[pl.ds(row0 + base, nr)])


def _sc_deg(dstp):
    mesh = plsc.VectorSubcoreMesh(core_axis_name="c", subcore_axis_name="s")
    return pl.kernel(
        _sc_deg_body,
        out_type=jax.ShapeDtypeStruct((NP, FW), jnp.float32),
        mesh=mesh,
        scratch_types=[
            pltpu.VMEM((16, CH), jnp.int32),
            pltpu.VMEM((128, FW), jnp.float32),
            pltpu.VMEM((128, FW), jnp.float32),
            pltpu.VMEM_SHARED((NP, FW), jnp.float32),
        ],
    )(dstp)


def _tc1_body(x_ref, w0_ref, b0_ref, deg_ref, us_ref, acc_ref):
    z = lax.dot_general(x_ref[...], w0_ref[...], (((1,), (1,)), ((), ())),
                        preferred_element_type=jnp.float32) + b0_ref[...]
    d = jnp.maximum(deg_ref[:, 0:1], 1.0)
    dinv = lax.rsqrt(d)
    rdeg = jnp.broadcast_to(1.0 / d, (NP, 16))
    dinvr = jnp.broadcast_to(dinv, (NP, 16))
    zeros48 = jnp.zeros((NP, 48), jnp.float32)
    zeros16 = jnp.zeros((NP, 16), jnp.float32)
    for h, sl in ((0, slice(0, 64)), (1, slice(64, 128))):
        zh = z[:, sl] * dinv
        us_ref[h * NP:(h + 1) * NP, 0:64] = zh
        us_ref[h * NP:(h + 1) * NP, 64:80] = dinvr
        us_ref[h * NP:(h + 1) * NP, 80:128] = zeros48
        acc_ref[h * NP:(h + 1) * NP, 0:64] = zh
        acc_ref[h * NP:(h + 1) * NP, 64:80] = dinvr
        acc_ref[h * NP:(h + 1) * NP, 80:96] = dinvr
        acc_ref[h * NP:(h + 1) * NP, 96:112] = rdeg
        acc_ref[h * NP:(h + 1) * NP, 112:128] = zeros16


def _tc1(x_pad, W0, b0r, degrep):
    full = lambda shape: pl.BlockSpec(shape, lambda: tuple(0 for _ in shape))
    return pl.pallas_call(
        _tc1_body,
        in_specs=[full((NP, F)), full((H, F)), full((1, H)), full((NP, FW))],
        out_specs=[full((2 * NP, FW)), full((2 * NP, FW))],
        out_shape=[
            jax.ShapeDtypeStruct((2 * NP, FW), jnp.float32),
            jax.ShapeDtypeStruct((2 * NP, FW), jnp.float32),
        ],
    )(x_pad, W0, b0r, degrep)


def _sc_body(usinit, accinit, srcp, dstp, ctab,
             acc_out, us,
             sbuf, dbuf, b0, b1, b2, ctb,
             outslab, sem, sg0, sg1, sg2, ss0, ss1, ss2):
    c = lax.axis_index("c")
    w = lax.axis_index("s")
    row0 = w * RPT
    zero16 = jnp.zeros((16,), jnp.float32)

    pltpu.sync_copy(ctab, ctb)

    for base, nr in _CHUNKS:
        pltpu.sync_copy(usinit.at[pl.ds(c * NP + row0 + base, nr)],
                        b0.at[pl.ds(0, nr)])
        pltpu.sync_copy(b0.at[pl.ds(0, nr)],
                        us.at[pl.ds(c * NP + row0 + base, nr)])
        pltpu.sync_copy(accinit.at[pl.ds(c * NP + row0 + base, nr)],
                        b1.at[pl.ds(0, nr)])
        pltpu.sync_copy(b1.at[pl.ds(0, nr)],
                        acc_out.at[pl.ds(c * NP + row0 + base, nr)])

    def _prop(k, _):
        def _zrow(i, _):
            for v in range(8):
                b0[i, pl.ds(16 * v, 16)] = zero16
            return 0
        lax.fori_loop(0, CH, _zrow, 0)
        for base, nr in _CHUNKS:
            pltpu.async_copy(b0.at[pl.ds(0, nr)],
                             outslab.at[pl.ds(row0 + base, nr)], sem)
        for base, nr in _CHUNKS:
            pltpu.make_async_copy(b0.at[pl.ds(0, nr)],
                                  outslab.at[pl.ds(row0 + base, nr)], sem).wait()
        plsc.subcore_barrier()

        def _g(sidx, j, buf, gsem):
            pltpu.async_copy(us.at[sidx.at[j]], buf, gsem)

        def _wg(sidx, buf, gsem):
            pltpu.make_async_copy(us.at[sidx.at[0]], buf, gsem).wait()

        def _s(didx, j, buf, ssem):
            pltpu.async_copy(buf, outslab.at[didx.at[j]], ssem, add=True)

        def _ws(didx, buf, ssem):
            pltpu.make_async_copy(buf, outslab.at[didx.at[0]], ssem).wait()

        coff = c * NP

        def _group(g, _):
            gg = (w * NG + g) * 16
            pltpu.sync_copy(srcp.at[pl.ds(gg, 16)], sbuf)
            pltpu.sync_copy(dstp.at[pl.ds(gg, 16)], dbuf)

            def _adjrow(i, _):
                for v in range(7):
                    sbuf[i, pl.ds(16 * v, 16)] = sbuf[i, pl.ds(16 * v, 16)] + coff
                return 0
            lax.fori_loop(0, 16, _adjrow, 0)

            _g(sbuf, 0, b0, sg0)
            _g(sbuf, 1, b1, sg1)
            _wg(sbuf, b0, sg0)
            _s(dbuf, 0, b0, ss0)
            _g(sbuf, 2, b2, sg2)
            _wg(sbuf, b1, sg1)
            _s(dbuf, 1, b1, ss1)
            _ws(dbuf, b0, ss0)
            _g(sbuf, 3, b0, sg0)

            def _tri(t, _):
                j = 3 * t + 2
                _wg(sbuf, b2, sg2)
                _s(dbuf, j, b2, ss2)
                _ws(dbuf, b1, ss1)
                _g(sbuf, j + 2, b1, sg1)
                _wg(sbuf, b0, sg0)
                _s(dbuf, j + 1, b0, ss0)
                _ws(dbuf, b2, ss2)
                _g(sbuf, j + 3, b2, sg2)
                _wg(sbuf, b1, sg1)
                _s(dbuf, j + 2, b1, ss1)
                _ws(dbuf, b0, ss0)
                _g(sbuf, j + 4, b0, sg0)
                return 0
            lax.fori_loop(0, 4, _tri, 0)
            _wg(sbuf, b2, sg2)
            _s(dbuf, 14, b2, ss2)
            _ws(dbuf, b1, ss1)
            _wg(sbuf, b0, sg0)
            _s(dbuf, 15, b0, ss0)
            _ws(dbuf, b2, ss2)
            _ws(dbuf, b0, ss0)
            return 0
        lax.fori_loop(0, NG, _group, 0)
        plsc.subcore_barrier()

        rowk = k % 16
        co = (k // 16) * 48
        c3v = ctb[rowk, pl.ds(co, 16)]
        c2v = ctb[rowk, pl.ds(co + 16, 16)]
        c1v = ctb[rowk, pl.ds(co + 32, 16)]
        for base, nr in _CHUNKS:
            pltpu.async_copy(outslab.at[pl.ds(row0 + base, nr)],
                             b0.at[pl.ds(0, nr)], sg0)
            pltpu.async_copy(acc_out.at[pl.ds(c * NP + row0 + base, nr)],
                             b1.at[pl.ds(0, nr)], sg1)
            pltpu.make_async_copy(outslab.at[pl.ds(row0 + base, nr)],
                                  b0.at[pl.ds(0, nr)], sg0).wait()
            pltpu.make_async_copy(acc_out.at[pl.ds(c * NP + row0 + base, nr)],
                                  b1.at[pl.ds(0, nr)], sg1).wait()

            def _post_row(r, _):
                rv = b1[r, pl.ds(96, 16)]
                for v in range(4):
                    t = b0[r, pl.ds(16 * v, 16)] * rv
                    b0[r, pl.ds(16 * v, 16)] = t
                    b1[r, pl.ds(16 * v, 16)] = b1[r, pl.ds(16 * v, 16)] + c3v * t
                t = b0[r, pl.ds(64, 16)] * rv
                b0[r, pl.ds(64, 16)] = t
                b1[r, pl.ds(64, 16)] = b1[r, pl.ds(64, 16)] + c2v * t
                b1[r, pl.ds(80, 16)] = b1[r, pl.ds(80, 16)] + c1v * t
                return 0
            lax.fori_loop(0, nr, _post_row, 0)
            pltpu.async_copy(b0.at[pl.ds(0, nr)],
                             us.at[pl.ds(c * NP + row0 + base, nr)], ss0)
            pltpu.async_copy(b1.at[pl.ds(0, nr)],
                             acc_out.at[pl.ds(c * NP + row0 + base, nr)], ss1)
            pltpu.make_async_copy(b0.at[pl.ds(0, nr)],
                                  us.at[pl.ds(c * NP + row0 + base, nr)], ss0).wait()
            pltpu.make_async_copy(b1.at[pl.ds(0, nr)],
                                  acc_out.at[pl.ds(c * NP + row0 + base, nr)], ss1).wait()
        return 0
    lax.fori_loop(1, K + 1, _prop, 0)


def _sc_call(usinit, accinit, srcp, dstp, ctab):
    mesh = plsc.VectorSubcoreMesh(core_axis_name="c", subcore_axis_name="s")
    f32 = jnp.float32
    return pl.kernel(
        _sc_body,
        out_type=(
            jax.ShapeDtypeStruct((2 * NP, FW), f32),
            jax.ShapeDtypeStruct((2 * NP, FW), f32),
        ),
        mesh=mesh,
        scratch_types=[
            pltpu.VMEM((16, CH), jnp.int32),
            pltpu.VMEM((16, CH), jnp.int32),
            pltpu.VMEM((CH, FW), f32),
            pltpu.VMEM((CH, FW), f32),
            pltpu.VMEM((CH, FW), f32),
            pltpu.VMEM((16, 128), f32),
            pltpu.VMEM_SHARED((NP, FW), f32),
            pltpu.SemaphoreType.DMA,
            pltpu.SemaphoreType.DMA,
            pltpu.SemaphoreType.DMA,
            pltpu.SemaphoreType.DMA,
            pltpu.SemaphoreType.DMA,
            pltpu.SemaphoreType.DMA,
            pltpu.SemaphoreType.DMA,
        ],
    )(usinit, accinit, srcp, dstp, ctab)


def _tc2_body(a0_ref, a1_ref, W1_ref, W2_ref, b1_ref, b2_ref,
              Wm0_ref, bm0_ref, Wm1_ref, bm1_ref, Wm2_ref, bm2_ref,
              Wm3_ref, bm3_ref, out_ref):
    dg = lambda a, b: lax.dot_general(a, b, (((1,), (1,)), ((), ())),
                                      preferred_element_type=jnp.float32)
    a0 = a0_ref[...]
    a1 = a1_ref[...]
    sq = lax.rsqrt(a0[:, 96:97])
    S = jnp.concatenate([a0[:, 0:64], a1[:, 0:64]], axis=1) * sq
    h = dg(dg(S, W1_ref[...]), W2_ref[...])
    w2b1 = dg(b1_ref[...], W2_ref[...])
    h = h + (a0[:, 64:65] * sq) * w2b1 + (a0[:, 80:81] * sq) * b2_ref[...]
    h = jnp.maximum(dg(h, Wm0_ref[...]) + bm0_ref[...], 0.0)
    h = jnp.maximum(dg(h, Wm1_ref[...]) + bm1_ref[...], 0.0)
    h = jnp.maximum(dg(h, Wm2_ref[...]) + bm2_ref[...], 0.0)
    lg = dg(h, Wm3_ref[...]) + bm3_ref[...]
    m = jnp.max(lg, axis=1, keepdims=True)
    s = jnp.sum(jnp.exp(lg - m), axis=1, keepdims=True)
    out_ref[...] = lg - m - jnp.log(s)


def _tc2(a0, a1, W1, W2, b1r, b2r, Wm0, bm0r, Wm1, bm1r, Wm2, bm2r, Wm3, bm3r):
    full = lambda shape: pl.BlockSpec(shape, lambda i: tuple(0 for _ in shape))
    return pl.pallas_call(
        _tc2_body,
        grid=(25,),
        in_specs=[
            pl.BlockSpec((400, FW), lambda i: (i, 0)),
            pl.BlockSpec((400, FW), lambda i: (i, 0)),
            full((H, H)), full((H, H)), full((1, H)), full((1, H)),
            full((H, H)), full((1, H)), full((H, H)), full((1, H)),
            full((H, H)), full((1, H)), full((C, H)), full((1, C)),
        ],
        out_specs=pl.BlockSpec((400, C), lambda i: (i, 0)),
        out_shape=jax.ShapeDtypeStruct((N, C), jnp.float32),
    )(a0, a1, W1, W2, b1r, b2r, Wm0, bm0r, Wm1, bm1r, Wm2, bm2r, Wm3, bm3r)


def kernel(x_in, edge_index, Wc0, bc0, Wc1, bc1, Wc2, bc2,
           Wm0, bm0, Wm1, bm1, Wm2, bm2, Wm3, bm3):
    f32 = jnp.float32
    src = edge_index[0]
    dst = edge_index[1]
    npad = EPAD - E
    pad_src = (jnp.arange(npad, dtype=jnp.int32) * 1009) % N
    pad_dst = N + (jnp.arange(npad, dtype=jnp.int32) % 16)
    srcp = jnp.concatenate([src, pad_src]).reshape(EPAD // CH, CH)
    dstp = jnp.concatenate([dst, pad_dst]).reshape(EPAD // CH, CH)

    degrep = _sc_deg(dstp)

    x_pad = jnp.concatenate([x_in, jnp.zeros((NP - N, F), f32)], axis=0)
    usinit, accinit = _tc1(x_pad, Wc0, bc0.reshape(1, H), degrep)

    ctab = _coef_table()
    acc, _ = _sc_call(usinit, accinit, srcp, dstp, ctab)

    out = _tc2(acc[0:NP], acc[NP:2 * NP],
               Wc1, Wc2, bc1.reshape(1, H), bc2.reshape(1, H),
               Wm0, bm0.reshape(1, H), Wm1, bm1.reshape(1, H),
               Wm2, bm2.reshape(1, H), Wm3, bm3.reshape(1, C))
    return out


# speedup vs baseline: 37.7418x; 1.0238x over previous
import functools
import math

import jax
import jax.numpy as jnp
from jax import lax
from jax.experimental import pallas as pl
from jax.experimental.pallas import tpu as pltpu
from jax.experimental.pallas import tpu_sc as plsc

N = 10000
F = 128
H = 128
C = 40
E = 320000
K = 20
NP = 10112
RPT = NP // 16
CH = 112
EPT = 21504
EPAD = EPT * 16
NG = 12
FW = 128
_CHUNKS = [(i * CH, min(CH, RPT - i * CH)) for i in range((RPT + CH - 1) // CH)]


def _coef_table():
    import numpy as np
    t = np.zeros((16, 128), np.float32)
    for k in range(K + 1):
        if k % 2 == 0:
            s = float((-1) ** (k // 2))
            co = 48 * (k // 16)
            t[k % 16, co + 0:co + 16] = s * 3.0 ** k / math.factorial(k)
            t[k % 16, co + 16:co + 32] = s * 2.0 ** k / math.factorial(k)
            t[k % 16, co + 32:co + 48] = s * 1.0 ** k / math.factorial(k)
    return jnp.asarray(t)


def _sc_deg_body(dstp, deg_out, dbuf, onesb, stg, degslab):
    c = lax.axis_index("c")
    w = lax.axis_index("s")
    row0 = w * RPT
    zero16 = jnp.zeros((16,), jnp.float32)
    one16 = jnp.ones((16,), jnp.float32)

    def _fill(i, _):
        for v in range(8):
            onesb[i, pl.ds(16 * v, 16)] = one16
            stg[i, pl.ds(16 * v, 16)] = zero16
        return 0
    lax.fori_loop(0, 128, _fill, 0)

    for base, nr in _CHUNKS:
        pltpu.sync_copy(stg.at[pl.ds(0, nr)],
                        degslab.at[pl.ds(row0 + base, nr)])
    plsc.subcore_barrier()

    def _group(g, _):
        gg = (w * NG + g) * 32 + 16
        pltpu.sync_copy(dstp.at[pl.ds(gg, 16)], dbuf)

        def _chunk(j, _):
            pltpu.sync_copy(onesb.at[pl.ds(0, CH)], degslab.at[dbuf.at[j]],
                            add=True)
            return 0
        lax.fori_loop(0, 16, _chunk, 0)
        return 0
    lax.fori_loop(0, NG, _group, 0)
    plsc.subcore_barrier()

    @pl.when(c == 0)
    def _():
        for base, nr in _CHUNKS:
            pltpu.sync_copy(degslab.at[pl.ds(row0 + base, nr)],
                            stg.at[pl.ds(0, nr)])
            pltpu.sync_copy(stg.at[pl.ds(0, nr)],
                            deg_out.at[pl.ds(row0 + base, nr)])


def _sc_deg(dstp):
    mesh = plsc.VectorSubcoreMesh(core_axis_name="c", subcore_axis_name="s")
    return pl.kernel(
        _sc_deg_body,
        out_type=jax.ShapeDtypeStruct((NP, FW), jnp.float32),
        mesh=mesh,
        scratch_types=[
            pltpu.VMEM((16, CH), jnp.int32),
            pltpu.VMEM((128, FW), jnp.float32),
            pltpu.VMEM((128, FW), jnp.float32),
            pltpu.VMEM_SHARED((NP, FW), jnp.float32),
        ],
    )(dstp)


def _tc1_body(x_ref, w0_ref, b0_ref, deg_ref, us_ref, acc_ref):
    z = lax.dot_general(x_ref[...], w0_ref[...], (((1,), (1,)), ((), ())),
                        preferred_element_type=jnp.float32) + b0_ref[...]
    d = jnp.maximum(deg_ref[:, 0:1], 1.0)
    dinv = lax.rsqrt(d)
    rdeg = jnp.broadcast_to(1.0 / d, (NP, 16))
    dinvr = jnp.broadcast_to(dinv, (NP, 16))
    zeros48 = jnp.zeros((NP, 48), jnp.float32)
    zeros16 = jnp.zeros((NP, 16), jnp.float32)
    for h, sl in ((0, slice(0, 64)), (1, slice(64, 128))):
        zh = z[:, sl] * dinv
        us_ref[h * NP:(h + 1) * NP, 0:64] = zh
        us_ref[h * NP:(h + 1) * NP, 64:80] = dinvr
        us_ref[h * NP:(h + 1) * NP, 80:128] = zeros48
        acc_ref[h * NP:(h + 1) * NP, 0:64] = zh
        acc_ref[h * NP:(h + 1) * NP, 64:80] = dinvr
        acc_ref[h * NP:(h + 1) * NP, 80:96] = dinvr
        acc_ref[h * NP:(h + 1) * NP, 96:112] = rdeg
        acc_ref[h * NP:(h + 1) * NP, 112:128] = zeros16


def _tc1(x_pad, W0, b0r, degrep):
    full = lambda shape: pl.BlockSpec(shape, lambda: tuple(0 for _ in shape))
    return pl.pallas_call(
        _tc1_body,
        in_specs=[full((NP, F)), full((H, F)), full((1, H)), full((NP, FW))],
        out_specs=[full((2 * NP, FW)), full((2 * NP, FW))],
        out_shape=[
            jax.ShapeDtypeStruct((2 * NP, FW), jnp.float32),
            jax.ShapeDtypeStruct((2 * NP, FW), jnp.float32),
        ],
    )(x_pad, W0, b0r, degrep)


def _sc_body(usinit, accinit, ixp, ctab,
             acc_out, us,
             ibuf, b0, b1, b2, ctb,
             outslab, sem, sg0, sg1, sg2, ss0, ss1, ss2):
    c = lax.axis_index("c")
    w = lax.axis_index("s")
    row0 = w * RPT
    zero16 = jnp.zeros((16,), jnp.float32)

    pltpu.sync_copy(ctab, ctb)

    for base, nr in _CHUNKS:
        pltpu.sync_copy(usinit.at[pl.ds(c * NP + row0 + base, nr)],
                        b0.at[pl.ds(0, nr)])
        pltpu.sync_copy(b0.at[pl.ds(0, nr)],
                        us.at[pl.ds(c * NP + row0 + base, nr)])
        pltpu.sync_copy(accinit.at[pl.ds(c * NP + row0 + base, nr)],
                        b1.at[pl.ds(0, nr)])
        pltpu.sync_copy(b1.at[pl.ds(0, nr)],
                        acc_out.at[pl.ds(c * NP + row0 + base, nr)])

    def _prop(k, _):
        def _zrow(i, _):
            for v in range(8):
                b0[i, pl.ds(16 * v, 16)] = zero16
            return 0
        lax.fori_loop(0, CH, _zrow, 0)
        for base, nr in _CHUNKS:
            pltpu.async_copy(b0.at[pl.ds(0, nr)],
                             outslab.at[pl.ds(row0 + base, nr)], sem)
        for base, nr in _CHUNKS:
            pltpu.make_async_copy(b0.at[pl.ds(0, nr)],
                                  outslab.at[pl.ds(row0 + base, nr)], sem).wait()
        plsc.subcore_barrier()

        def _g(sidx, j, buf, gsem):
            pltpu.async_copy(us.at[sidx.at[j]], buf, gsem)

        def _wg(sidx, buf, gsem):
            pltpu.make_async_copy(us.at[sidx.at[0]], buf, gsem).wait()

        def _s(didx, j, buf, ssem):
            pltpu.async_copy(buf, outslab.at[didx.at[16 + j]], ssem, add=True)

        def _ws(didx, buf, ssem):
            pltpu.make_async_copy(buf, outslab.at[didx.at[16]], ssem).wait()

        coff = c * NP

        def _group(g, _):
            gg = (w * NG + g) * 32
            pltpu.sync_copy(ixp.at[pl.ds(gg, 32)], ibuf)

            def _adjrow(i, _):
                for v in range(7):
                    ibuf[i, pl.ds(16 * v, 16)] = ibuf[i, pl.ds(16 * v, 16)] + coff
                return 0
            lax.fori_loop(0, 16, _adjrow, 0)

            _g(ibuf, 0, b0, sg0)
            _g(ibuf, 1, b1, sg1)
            _wg(ibuf, b0, sg0)
            _s(ibuf, 0, b0, ss0)
            _g(ibuf, 2, b2, sg2)
            _wg(ibuf, b1, sg1)
            _s(ibuf, 1, b1, ss1)
            _ws(ibuf, b0, ss0)
            _g(ibuf, 3, b0, sg0)

            def _tri(t, _):
                j = 3 * t + 2
                _wg(ibuf, b2, sg2)
                _s(ibuf, j, b2, ss2)
                _ws(ibuf, b1, ss1)
                _g(ibuf, j + 2, b1, sg1)
                _wg(ibuf, b0, sg0)
                _s(ibuf, j + 1, b0, ss0)
                _ws(ibuf, b2, ss2)
                _g(ibuf, j + 3, b2, sg2)
                _wg(ibuf, b1, sg1)
                _s(ibuf, j + 2, b1, ss1)
                _ws(ibuf, b0, ss0)
                _g(ibuf, j + 4, b0, sg0)
                return 0
            lax.fori_loop(0, 4, _tri, 0)
            _wg(ibuf, b2, sg2)
            _s(ibuf, 14, b2, ss2)
            _ws(ibuf, b1, ss1)
            _wg(ibuf, b0, sg0)
            _s(ibuf, 15, b0, ss0)
            _ws(ibuf, b2, ss2)
            _ws(ibuf, b0, ss0)
            return 0
        lax.fori_loop(0, NG, _group, 0)
        plsc.subcore_barrier()

        rowk = k % 16
        co = (k // 16) * 48
        c3v = ctb[rowk, pl.ds(co, 16)]
        c2v = ctb[rowk, pl.ds(co + 16, 16)]
        c1v = ctb[rowk, pl.ds(co + 32, 16)]
        for base, nr in _CHUNKS:
            pltpu.async_copy(outslab.at[pl.ds(row0 + base, nr)],
                             b0.at[pl.ds(0, nr)], sg0)
            pltpu.async_copy(acc_out.at[pl.ds(c * NP + row0 + base, nr)],
                             b1.at[pl.ds(0, nr)], sg1)
            pltpu.make_async_copy(outslab.at[pl.ds(row0 + base, nr)],
                                  b0.at[pl.ds(0, nr)], sg0).wait()
            pltpu.make_async_copy(acc_out.at[pl.ds(c * NP + row0 + base, nr)],
                                  b1.at[pl.ds(0, nr)], sg1).wait()

            def _post_row(r, _):
                rv = b1[r, pl.ds(96, 16)]
                for v in range(4):
                    t = b0[r, pl.ds(16 * v, 16)] * rv
                    b0[r, pl.ds(16 * v, 16)] = t
                    b1[r, pl.ds(16 * v, 16)] = b1[r, pl.ds(16 * v, 16)] + c3v * t
                t = b0[r, pl.ds(64, 16)] * rv
                b0[r, pl.ds(64, 16)] = t
                b1[r, pl.ds(64, 16)] = b1[r, pl.ds(64, 16)] + c2v * t
                b1[r, pl.ds(80, 16)] = b1[r, pl.ds(80, 16)] + c1v * t
                return 0
            lax.fori_loop(0, nr, _post_row, 0)
            pltpu.async_copy(b0.at[pl.ds(0, nr)],
                             us.at[pl.ds(c * NP + row0 + base, nr)], ss0)
            pltpu.async_copy(b1.at[pl.ds(0, nr)],
                             acc_out.at[pl.ds(c * NP + row0 + base, nr)], ss1)
            pltpu.make_async_copy(b0.at[pl.ds(0, nr)],
                                  us.at[pl.ds(c * NP + row0 + base, nr)], ss0).wait()
            pltpu.make_async_copy(b1.at[pl.ds(0, nr)],
                                  acc_out.at[pl.ds(c * NP + row0 + base, nr)], ss1).wait()
        return 0
    lax.fori_loop(1, K + 1, _prop, 0)


def _sc_call(usinit, accinit, ixp, ctab):
    mesh = plsc.VectorSubcoreMesh(core_axis_name="c", subcore_axis_name="s")
    f32 = jnp.float32
    return pl.kernel(
        _sc_body,
        out_type=(
            jax.ShapeDtypeStruct((2 * NP, FW), f32),
            jax.ShapeDtypeStruct((2 * NP, FW), f32),
        ),
        mesh=mesh,
        scratch_types=[
            pltpu.VMEM((32, CH), jnp.int32),
            pltpu.VMEM((CH, FW), f32),
            pltpu.VMEM((CH, FW), f32),
            pltpu.VMEM((CH, FW), f32),
            pltpu.VMEM((16, 128), f32),
            pltpu.VMEM_SHARED((NP, FW), f32),
            pltpu.SemaphoreType.DMA,
            pltpu.SemaphoreType.DMA,
            pltpu.SemaphoreType.DMA,
            pltpu.SemaphoreType.DMA,
            pltpu.SemaphoreType.DMA,
            pltpu.SemaphoreType.DMA,
            pltpu.SemaphoreType.DMA,
        ],
    )(usinit, accinit, ixp, ctab)


def _tc2_body(a0_ref, a1_ref, W1_ref, W2_ref, b1_ref, b2_ref,
              Wm0_ref, bm0_ref, Wm1_ref, bm1_ref, Wm2_ref, bm2_ref,
              Wm3_ref, bm3_ref, out_ref):
    dg = lambda a, b: lax.dot_general(a, b, (((1,), (1,)), ((), ())),
                                      preferred_element_type=jnp.float32)
    a0 = a0_ref[...]
    a1 = a1_ref[...]
    sq = lax.rsqrt(a0[:, 96:97])
    S = jnp.concatenate([a0[:, 0:64], a1[:, 0:64]], axis=1) * sq
    h = dg(dg(S, W1_ref[...]), W2_ref[...])
    w2b1 = dg(b1_ref[...], W2_ref[...])
    h = h + (a0[:, 64:65] * sq) * w2b1 + (a0[:, 80:81] * sq) * b2_ref[...]
    h = jnp.maximum(dg(h, Wm0_ref[...]) + bm0_ref[...], 0.0)
    h = jnp.maximum(dg(h, Wm1_ref[...]) + bm1_ref[...], 0.0)
    h = jnp.maximum(dg(h, Wm2_ref[...]) + bm2_ref[...], 0.0)
    lg = dg(h, Wm3_ref[...]) + bm3_ref[...]
    m = jnp.max(lg, axis=1, keepdims=True)
    s = jnp.sum(jnp.exp(lg - m), axis=1, keepdims=True)
    out_ref[...] = lg - m - jnp.log(s)


def _tc2(a0, a1, W1, W2, b1r, b2r, Wm0, bm0r, Wm1, bm1r, Wm2, bm2r, Wm3, bm3r):
    full = lambda shape: pl.BlockSpec(shape, lambda i: tuple(0 for _ in shape))
    return pl.pallas_call(
        _tc2_body,
        grid=(25,),
        in_specs=[
            pl.BlockSpec((400, FW), lambda i: (i, 0)),
            pl.BlockSpec((400, FW), lambda i: (i, 0)),
            full((H, H)), full((H, H)), full((1, H)), full((1, H)),
            full((H, H)), full((1, H)), full((H, H)), full((1, H)),
            full((H, H)), full((1, H)), full((C, H)), full((1, C)),
        ],
        out_specs=pl.BlockSpec((400, C), lambda i: (i, 0)),
        out_shape=jax.ShapeDtypeStruct((N, C), jnp.float32),
    )(a0, a1, W1, W2, b1r, b2r, Wm0, bm0r, Wm1, bm1r, Wm2, bm2r, Wm3, bm3r)


def kernel(x_in, edge_index, Wc0, bc0, Wc1, bc1, Wc2, bc2,
           Wm0, bm0, Wm1, bm1, Wm2, bm2, Wm3, bm3):
    f32 = jnp.float32
    src = edge_index[0]
    dst = edge_index[1]
    npad = EPAD - E
    pad_src = (jnp.arange(npad, dtype=jnp.int32) * 1009) % N
    pad_dst = N + (jnp.arange(npad, dtype=jnp.int32) % 16)
    srcp = jnp.concatenate([src, pad_src]).reshape(EPAD // (16 * CH), 16, CH)
    dstp = jnp.concatenate([dst, pad_dst]).reshape(EPAD // (16 * CH), 16, CH)
    ixp = jnp.concatenate([srcp, dstp], axis=1).reshape(2 * EPAD // CH, CH)

    degrep = _sc_deg(ixp)

    x_pad = jnp.concatenate([x_in, jnp.zeros((NP - N, F), f32)], axis=0)
    usinit, accinit = _tc1(x_pad, Wc0, bc0.reshape(1, H), degrep)

    ctab = _coef_table()
    acc, _ = _sc_call(usinit, accinit, ixp, ctab)

    out = _tc2(acc[0:NP], acc[NP:2 * NP],
               Wc1, Wc2, bc1.reshape(1, H), bc2.reshape(1, H),
               Wm0, bm0.reshape(1, H), Wm1, bm1.reshape(1, H),
               Wm2, bm2.reshape(1, H), Wm3, bm3.reshape(1, C))
    return out
